# Initial kernel scaffold; baseline (speedup 1.0000x reference)
import jax
import jax.numpy as jnp
from jax.experimental import pallas as pl


def kernel(coordinate, size, x_embedding, y_embedding):
    raise NotImplementedError("write your pallas kernel here")


# speedup vs baseline: 2.1134x; 2.1134x over previous
import functools

import jax
import jax.numpy as jnp
from jax import lax
from jax.experimental import pallas as pl
from jax.experimental.pallas import tpu as pltpu
from jax.experimental.pallas import tpu_sc as plsc

RES_X = 1024
RES_Y = 1024
D_HALF = 256
L = 16
NC, NS = 2, 16
NW = NC * NS
B = 16 * 32 * 128
TPW = B // NW
CHUNK_ROWS = 128
NCHUNK = (2 * TPW) // CHUNK_ROWS

Y_OFF = RES_X + 1
ZERO_X = RES_X
ZERO_Y = Y_OFF + RES_Y

_mesh = plsc.VectorSubcoreMesh(core_axis_name="c", subcore_axis_name="s")


@functools.partial(
    pl.kernel,
    out_type=jax.ShapeDtypeStruct((2 * B, D_HALF), jnp.float32),
    mesh=_mesh,
    scratch_types=[
        pltpu.VMEM((2 * TPW,), jnp.float32),
        pltpu.VMEM((L,), jnp.float32),
        pltpu.VMEM((2 * TPW,), jnp.int32),
        pltpu.VMEM((CHUNK_ROWS, D_HALF), jnp.float32),
        pltpu.VMEM((CHUNK_ROWS, D_HALF), jnp.float32),
        pltpu.SemaphoreType.DMA,
        pltpu.SemaphoreType.DMA,
    ],
)
def _pos_lookup(coord_hbm, size_hbm, table_hbm, out_hbm,
                coord_v, size_v, idx_v, rows0, rows1, sem0, sem1):
    wid = lax.axis_index("s") * NC + lax.axis_index("c")
    tbase = wid * TPW

    pltpu.sync_copy(coord_hbm.at[pl.ds(2 * tbase, 2 * TPW)], coord_v)
    pltpu.sync_copy(size_hbm, size_v)

    s_vec = size_v[pl.ds(0, L)]
    iota = lax.iota(jnp.int32, L)
    parity = iota & 1
    off_vec = parity * Y_OFF
    zero_vec = off_vec + ZERO_X
    evens = iota & ~1
    dnums = lax.GatherDimensionNumbers(
        offset_dims=(), collapsed_slice_dims=(0,), start_index_map=(0,))

    def index_body(t, _):
        v = coord_v[pl.ds(t * L, L)]
        f = (jnp.float32(RES_X) * v) / s_vec
        idx = jnp.clip(f.astype(jnp.int32), 0, RES_X - 1) + off_vec
        xboth = lax.gather(v, evens[:, None], dnums, (1,),
                           mode=lax.GatherScatterMode.PROMISE_IN_BOUNDS)
        idx = jnp.where(xboth < 0.0, zero_vec, idx)
        idx_v[pl.ds(t * L, L)] = idx
        return 0

    lax.fori_loop(0, (2 * TPW) // L, index_body, 0)

    obase = 2 * tbase

    def idx_slice(k):
        return idx_v.at[pl.ds(k * CHUNK_ROWS, CHUNK_ROWS)]

    pltpu.async_copy(table_hbm.at[idx_slice(0)], rows0, sem0)

    def gather_body(kk, _):
        k0 = kk * 2
        c1 = pltpu.async_copy(table_hbm.at[idx_slice(k0 + 1)], rows1, sem1)
        pltpu.make_async_copy(table_hbm.at[idx_slice(k0)], rows0, sem0).wait()
        pltpu.sync_copy(rows0, out_hbm.at[pl.ds(obase + k0 * CHUNK_ROWS,
                                                CHUNK_ROWS)])

        @pl.when(kk + 1 < NCHUNK // 2)
        def _():
            pltpu.async_copy(table_hbm.at[idx_slice(k0 + 2)], rows0, sem0)

        c1.wait()
        pltpu.sync_copy(rows1, out_hbm.at[pl.ds(obase + (k0 + 1) * CHUNK_ROWS,
                                                CHUNK_ROWS)])
        return 0

    lax.fori_loop(0, NCHUNK // 2, gather_body, 0)


def kernel(coordinate, size, x_embedding, y_embedding):
    coord_flat = coordinate.reshape(2 * B)
    zrow = jnp.zeros((1, D_HALF), jnp.float32)
    table = jnp.concatenate([x_embedding, zrow, y_embedding, zrow], axis=0)
    sizes = jnp.tile(size[jnp.array([1, 0])], L // 2)
    out = _pos_lookup(coord_flat, sizes, table)
    return out.reshape(16, 32, 128, 2 * D_HALF)
